# baseline (speedup 1.0000x reference)
import jax
import jax.numpy as jnp
from jax import lax
from jax.experimental import pallas as pl
from jax.experimental.pallas import tpu as pltpu
from jax.experimental.pallas import tpu_sc as plsc

ROWS = 16384
COLS = 50
DIM = 32
SCALE = 10.0
LANES = 128

NC = 2
NS = 16
NW = NC * NS
IBLK = ROWS // LANES
IB_PER_W = IBLK // NW


def _sc_body(xf_hbm, w_hbm, out5_hbm, xblk, cidx, gbuf, tbuf, gsem):
    wid = lax.axis_index("s") * NC + lax.axis_index("c")
    iota = lax.iota(jnp.int32, 16)

    def iblk_body(ib, carry):
        c = wid * IB_PER_W + ib
        pltpu.sync_copy(
            xf_hbm.at[pl.ds(c * LANES * LANES, LANES * LANES)], xblk)

        def j1_body(j1, carry2):
            for k in range(8):
                flat = (k * 16 + iota) * LANES + j1
                cidx[pl.ds(k * 16, 16)] = plsc.load_gather(xblk, [flat])
            pltpu.async_copy(w_hbm.at[cidx], gbuf, gsem).wait()

            def d_body(d, carry3):
                dvec = jnp.full((16,), 0, jnp.int32) + d
                for k in range(8):
                    rows16 = k * 16 + iota
                    vals = plsc.load_gather(gbuf, [rows16, dvec])
                    tbuf[d, pl.ds(k * 16, 16)] = vals * SCALE
                return carry3

            lax.fori_loop(0, DIM, d_body, 0)

            for r in range(4):
                pltpu.sync_copy(tbuf.at[pl.ds(r * 8, 8), :],
                                out5_hbm.at[j1, r, c])
            return carry2

        lax.fori_loop(0, COLS, j1_body, 0)
        return carry

    lax.fori_loop(0, IB_PER_W, iblk_body, 0)


def kernel(x, weight):
    xf = jnp.pad(x.astype(jnp.int32), ((0, 0), (0, LANES - COLS))).reshape(-1)
    mesh = plsc.VectorSubcoreMesh(core_axis_name="c", subcore_axis_name="s")
    out5 = pl.kernel(
        _sc_body,
        out_type=jax.ShapeDtypeStruct((COLS, DIM // 8, IBLK, 8, LANES),
                                      jnp.float32),
        mesh=mesh,
        scratch_types=[
            pltpu.VMEM((LANES * LANES,), jnp.int32),
            pltpu.VMEM((LANES,), jnp.int32),
            pltpu.VMEM((LANES, DIM), jnp.float32),
            pltpu.VMEM((DIM, LANES), jnp.float32),
            pltpu.SemaphoreType.DMA,
        ],
        compiler_params=pltpu.CompilerParams(use_tc_tiling_on_sc=False,
                                             needs_layout_passes=False),
    )(xf, weight)
    return out5.transpose((2, 4, 0, 1, 3)).reshape(ROWS, COLS, DIM)


# speedup vs baseline: 1.1866x; 1.1866x over previous
import jax
import jax.numpy as jnp
from jax import lax
from jax.experimental import pallas as pl
from jax.experimental.pallas import tpu as pltpu
from jax.experimental.pallas import tpu_sc as plsc

ROWS = 16384
COLS = 50
DIM = 32
SCALE = 10.0
LANES = 128

NC = 2
NS = 16
NW = NC * NS
IBLK = ROWS // LANES
IB_PER_W = IBLK // NW
PAIRS = COLS // 2


def _sc_body(xf_hbm, w_hbm, out5_hbm, xblk,
             cidx0, cidx1, gbuf0, gbuf1, tbuf0, tbuf1,
             gsem0, gsem1, osem0, osem1):
    wid = lax.axis_index("s") * NC + lax.axis_index("c")
    iota = lax.iota(jnp.int32, 16)
    cidx = (cidx0, cidx1)
    gbuf = (gbuf0, gbuf1)
    tbuf = (tbuf0, tbuf1)
    gsem = (gsem0, gsem1)
    osem = (osem0, osem1)

    def build_cidx(s, j1):
        for k in range(8):
            flat = (k * 16 + iota) * LANES + j1
            cidx[s][pl.ds(k * 16, 16)] = plsc.load_gather(xblk, [flat])

    def fire_gather(s):
        pltpu.async_copy(w_hbm.at[cidx[s]], gbuf[s], gsem[s])

    def wait_gather(s):
        pltpu.make_async_copy(w_hbm.at[cidx[s]], gbuf[s], gsem[s]).wait()

    def transpose(s):
        def d_body(d, c3):
            dvec = jnp.full((16,), 0, jnp.int32) + d
            for k in range(8):
                vals = plsc.load_gather(gbuf[s], [k * 16 + iota, dvec])
                tbuf[s][d, pl.ds(k * 16, 16)] = vals * SCALE
            return c3

        lax.fori_loop(0, DIM, d_body, 0)

    def fire_out(s, j1, c):
        for r in range(4):
            pltpu.async_copy(tbuf[s].at[pl.ds(r * 8, 8), :],
                             out5_hbm.at[j1, r, c], osem[s])

    def drain_out(s):
        for r in range(4):
            pltpu.make_async_copy(tbuf[s].at[pl.ds(r * 8, 8), :],
                                  out5_hbm.at[0, 0, 0], osem[s]).wait()

    def iblk_body(ib, carry):
        c = wid * IB_PER_W + ib
        pltpu.sync_copy(
            xf_hbm.at[pl.ds(c * LANES * LANES, LANES * LANES)], xblk)
        build_cidx(0, 0)
        fire_gather(0)

        def pair(t, carry2):
            j1a = 2 * t
            j1b = 2 * t + 1
            build_cidx(1, j1b)
            fire_gather(1)
            wait_gather(0)

            @pl.when(t > 0)
            def _():
                drain_out(0)

            transpose(0)
            fire_out(0, j1a, c)

            @pl.when(t < PAIRS - 1)
            def _():
                build_cidx(0, j1a + 2)
                fire_gather(0)

            wait_gather(1)

            @pl.when(t > 0)
            def _():
                drain_out(1)

            transpose(1)
            fire_out(1, j1b, c)
            return carry2

        lax.fori_loop(0, PAIRS, pair, 0)
        drain_out(0)
        drain_out(1)
        return carry

    lax.fori_loop(0, IB_PER_W, iblk_body, 0)


def kernel(x, weight):
    xf = jnp.pad(x.astype(jnp.int32), ((0, 0), (0, LANES - COLS))).reshape(-1)
    mesh = plsc.VectorSubcoreMesh(core_axis_name="c", subcore_axis_name="s")
    out5 = pl.kernel(
        _sc_body,
        out_type=jax.ShapeDtypeStruct((COLS, DIM // 8, IBLK, 8, LANES),
                                      jnp.float32),
        mesh=mesh,
        scratch_types=[
            pltpu.VMEM((LANES * LANES,), jnp.int32),
            pltpu.VMEM((LANES,), jnp.int32),
            pltpu.VMEM((LANES,), jnp.int32),
            pltpu.VMEM((LANES, DIM), jnp.float32),
            pltpu.VMEM((LANES, DIM), jnp.float32),
            pltpu.VMEM((DIM, LANES), jnp.float32),
            pltpu.VMEM((DIM, LANES), jnp.float32),
            pltpu.SemaphoreType.DMA,
            pltpu.SemaphoreType.DMA,
            pltpu.SemaphoreType.DMA,
            pltpu.SemaphoreType.DMA,
        ],
        compiler_params=pltpu.CompilerParams(use_tc_tiling_on_sc=False,
                                             needs_layout_passes=False),
    )(xf, weight)
    return out5.transpose((2, 4, 0, 1, 3)).reshape(ROWS, COLS, DIM)


# speedup vs baseline: 1.2587x; 1.0608x over previous
import jax
import jax.numpy as jnp
from jax import lax
from jax.experimental import pallas as pl
from jax.experimental.pallas import tpu as pltpu
from jax.experimental.pallas import tpu_sc as plsc

ROWS = 16384
NUM_ROWS_W = 1000000
COLS = 50
DIM = 32
SCALE = 10.0
LANES = 128

NC = 2
NS = 16
NW = NC * NS
IBLK = ROWS // LANES
IB_PER_W = IBLK // NW
PAIRS = COLS // 2


def _sc_body(xf_hbm, w_hbm, out5_hbm, xblk,
             cidx0, cidx1, gbuf0, gbuf1, tbuf0, tbuf1,
             gsem0, gsem1, osem0, osem1):
    wid = lax.axis_index("s") * NC + lax.axis_index("c")
    iota = lax.iota(jnp.int32, 16)
    cidx = (cidx0, cidx1)
    gbuf = (gbuf0, gbuf1)
    tbuf = (tbuf0, tbuf1)
    gsem = (gsem0, gsem1)
    osem = (osem0, osem1)

    def build_cidx(s, j1):
        for k in range(8):
            flat = (k * 16 + iota) * LANES + j1
            v = plsc.load_gather(xblk, [flat])
            g = (v & ~2047) + ((v & 511) << 2) + ((v >> 9) & 3)
            cidx[s][pl.ds(k * 16, 16)] = g

    def fire_gather(s):
        pltpu.async_copy(w_hbm.at[cidx[s]], gbuf[s], gsem[s])

    def wait_gather(s):
        pltpu.make_async_copy(w_hbm.at[cidx[s]], gbuf[s], gsem[s]).wait()

    def transpose(s):
        def d_body(d, c3):
            dvec = jnp.full((16,), 0, jnp.int32) + d
            for k in range(8):
                vals = plsc.load_gather(gbuf[s], [k * 16 + iota, dvec])
                tbuf[s][d, pl.ds(k * 16, 16)] = vals * SCALE
            return c3

        lax.fori_loop(0, DIM, d_body, 0)

    def fire_out(s, j1, c):
        for r in range(4):
            pltpu.async_copy(tbuf[s].at[pl.ds(r * 8, 8), :],
                             out5_hbm.at[j1, r, c], osem[s])

    def drain_out(s):
        for r in range(4):
            pltpu.make_async_copy(tbuf[s].at[pl.ds(r * 8, 8), :],
                                  out5_hbm.at[0, 0, 0], osem[s]).wait()

    def iblk_body(ib, carry):
        c = wid * IB_PER_W + ib
        pltpu.sync_copy(
            xf_hbm.at[pl.ds(c * LANES * LANES, LANES * LANES)], xblk)
        build_cidx(0, 0)
        fire_gather(0)

        def pair(t, carry2):
            j1a = 2 * t
            j1b = 2 * t + 1
            build_cidx(1, j1b)
            fire_gather(1)
            wait_gather(0)

            @pl.when(t > 0)
            def _():
                drain_out(0)

            transpose(0)
            fire_out(0, j1a, c)

            @pl.when(t < PAIRS - 1)
            def _():
                build_cidx(0, j1a + 2)
                fire_gather(0)

            wait_gather(1)

            @pl.when(t > 0)
            def _():
                drain_out(1)

            transpose(1)
            fire_out(1, j1b, c)
            return carry2

        lax.fori_loop(0, PAIRS, pair, 0)
        drain_out(0)
        drain_out(1)
        return carry

    lax.fori_loop(0, IB_PER_W, iblk_body, 0)


TBLK = 2048
TGRID = -(-NUM_ROWS_W // TBLK)
WROWS = TGRID * TBLK


def _tc_relayout_body(wt_ref, out_ref):
    for m in range(4):
        out_ref[:, 32 * m:32 * (m + 1)] = (
            wt_ref[:, 512 * m:512 * (m + 1)].T)


def _relayout_weight(weight):
    w_rm = pl.pallas_call(
        _tc_relayout_body,
        grid=(TGRID,),
        in_specs=[pl.BlockSpec((DIM, TBLK), lambda i: (0, i))],
        out_specs=pl.BlockSpec((TBLK // 4, LANES), lambda i: (i, 0)),
        out_shape=jax.ShapeDtypeStruct((WROWS // 4, LANES), jnp.float32),
    )(weight.T)
    return w_rm.reshape(WROWS, DIM)


def kernel(x, weight):
    xf = jnp.pad(x.astype(jnp.int32), ((0, 0), (0, LANES - COLS))).reshape(-1)
    w_lin = _relayout_weight(weight)
    mesh = plsc.VectorSubcoreMesh(core_axis_name="c", subcore_axis_name="s")
    out5 = pl.kernel(
        _sc_body,
        out_type=jax.ShapeDtypeStruct((COLS, DIM // 8, IBLK, 8, LANES),
                                      jnp.float32),
        mesh=mesh,
        scratch_types=[
            pltpu.VMEM((LANES * LANES,), jnp.int32),
            pltpu.VMEM((LANES,), jnp.int32),
            pltpu.VMEM((LANES,), jnp.int32),
            pltpu.VMEM((LANES, DIM), jnp.float32),
            pltpu.VMEM((LANES, DIM), jnp.float32),
            pltpu.VMEM((DIM, LANES), jnp.float32),
            pltpu.VMEM((DIM, LANES), jnp.float32),
            pltpu.SemaphoreType.DMA,
            pltpu.SemaphoreType.DMA,
            pltpu.SemaphoreType.DMA,
            pltpu.SemaphoreType.DMA,
        ],
        compiler_params=pltpu.CompilerParams(use_tc_tiling_on_sc=False,
                                             needs_layout_passes=False),
    )(xf, w_lin)
    return out5.transpose((2, 4, 0, 1, 3)).reshape(ROWS, COLS, DIM)


# speedup vs baseline: 2.1076x; 1.6744x over previous
import jax
import jax.numpy as jnp
from jax import lax
from jax.experimental import pallas as pl
from jax.experimental.pallas import tpu as pltpu
from jax.experimental.pallas import tpu_sc as plsc

ROWS = 16384
NUM_ROWS_W = 1000000
COLS = 50
DIM = 32
SCALE = 10.0
LANES = 128

NC = 2
NS = 16
NW = NC * NS
IBLK = ROWS // LANES
IB_PER_W = IBLK // NW
PAIRS = COLS // 2


def _sc_body(xf_hbm, w_hbm, out5_hbm, xblk,
             cidx0, cidx1, gbuf0, gbuf1, tbuf0, tbuf1,
             gsem0, gsem1, osem0, osem1):
    wid = lax.axis_index("s") * NC + lax.axis_index("c")
    iota = lax.iota(jnp.int32, 16)
    cidx = (cidx0, cidx1)
    gbuf = (gbuf0, gbuf1)
    tbuf = (tbuf0, tbuf1)
    gsem = (gsem0, gsem1)
    osem = (osem0, osem1)

    def build_cidx(s, j1):
        for k in range(8):
            flat = (k * 16 + iota) * LANES + j1
            v = plsc.load_gather(xblk, [flat])
            g = (v & ~2047) + ((v & 511) << 2) + ((v >> 9) & 3)
            cidx[s][pl.ds(k * 16, 16)] = g

    def fire_gather(s):
        pltpu.async_copy(w_hbm.at[cidx[s]], gbuf[s], gsem[s])

    def wait_gather(s):
        pltpu.make_async_copy(w_hbm.at[cidx[s]], gbuf[s], gsem[s]).wait()

    def transpose(s):
        def r_body(r, c3):
            rvec = jnp.full((16,), 0, jnp.int32) + r
            for h in range(2):
                vals = gbuf[s][r, pl.ds(16 * h, 16)]
                plsc.store_scatter(tbuf[s], [16 * h + iota, rvec], vals)
            return c3

        lax.fori_loop(0, LANES, r_body, 0)

    def fire_out(s, j1, c):
        for r in range(4):
            pltpu.async_copy(tbuf[s].at[pl.ds(r * 8, 8), pl.ds(0, LANES)],
                             out5_hbm.at[j1, r, c], osem[s])

    def drain_out(s):
        for r in range(4):
            pltpu.make_async_copy(
                tbuf[s].at[pl.ds(r * 8, 8), pl.ds(0, LANES)],
                out5_hbm.at[0, 0, 0], osem[s]).wait()

    def iblk_body(ib, carry):
        c = wid * IB_PER_W + ib
        pltpu.sync_copy(
            xf_hbm.at[pl.ds(c * LANES * LANES, LANES * LANES)], xblk)
        build_cidx(0, 0)
        fire_gather(0)

        def pair(t, carry2):
            j1a = 2 * t
            j1b = 2 * t + 1
            build_cidx(1, j1b)
            fire_gather(1)
            wait_gather(0)

            @pl.when(t > 0)
            def _():
                drain_out(0)

            transpose(0)
            fire_out(0, j1a, c)

            @pl.when(t < PAIRS - 1)
            def _():
                build_cidx(0, j1a + 2)
                fire_gather(0)

            wait_gather(1)

            @pl.when(t > 0)
            def _():
                drain_out(1)

            transpose(1)
            fire_out(1, j1b, c)
            return carry2

        lax.fori_loop(0, PAIRS, pair, 0)
        drain_out(0)
        drain_out(1)
        return carry

    lax.fori_loop(0, IB_PER_W, iblk_body, 0)


TBLK = 2048
TGRID = -(-NUM_ROWS_W // TBLK)
WROWS = TGRID * TBLK


def _tc_relayout_body(wt_ref, out_ref):
    for m in range(4):
        out_ref[:, 32 * m:32 * (m + 1)] = (
            wt_ref[:, 512 * m:512 * (m + 1)].T * SCALE)


def _relayout_weight(weight):
    w_rm = pl.pallas_call(
        _tc_relayout_body,
        grid=(TGRID,),
        in_specs=[pl.BlockSpec((DIM, TBLK), lambda i: (0, i))],
        out_specs=pl.BlockSpec((TBLK // 4, LANES), lambda i: (i, 0)),
        out_shape=jax.ShapeDtypeStruct((WROWS // 4, LANES), jnp.float32),
    )(weight.T)
    return w_rm.reshape(WROWS, DIM)


def kernel(x, weight):
    xf = jnp.pad(x.astype(jnp.int32), ((0, 0), (0, LANES - COLS))).reshape(-1)
    w_lin = _relayout_weight(weight)
    mesh = plsc.VectorSubcoreMesh(core_axis_name="c", subcore_axis_name="s")
    out5 = pl.kernel(
        _sc_body,
        out_type=jax.ShapeDtypeStruct((COLS, DIM // 8, IBLK, 8, LANES),
                                      jnp.float32),
        mesh=mesh,
        scratch_types=[
            pltpu.VMEM((LANES * LANES,), jnp.int32),
            pltpu.VMEM((LANES,), jnp.int32),
            pltpu.VMEM((LANES,), jnp.int32),
            pltpu.VMEM((LANES, DIM), jnp.float32),
            pltpu.VMEM((LANES, DIM), jnp.float32),
            pltpu.VMEM((DIM, LANES + 1), jnp.float32),
            pltpu.VMEM((DIM, LANES + 1), jnp.float32),
            pltpu.SemaphoreType.DMA,
            pltpu.SemaphoreType.DMA,
            pltpu.SemaphoreType.DMA,
            pltpu.SemaphoreType.DMA,
        ],
        compiler_params=pltpu.CompilerParams(use_tc_tiling_on_sc=False,
                                             needs_layout_passes=False),
    )(xf, w_lin)
    return out5.transpose((2, 4, 0, 1, 3)).reshape(ROWS, COLS, DIM)


# speedup vs baseline: 2.7566x; 1.3080x over previous
import jax
import jax.numpy as jnp
from jax import lax
from jax.experimental import pallas as pl
from jax.experimental.pallas import tpu as pltpu
from jax.experimental.pallas import tpu_sc as plsc

ROWS = 16384
NUM_ROWS_W = 1000000
COLS = 50
DIM = 32
SCALE = 10.0
LANES = 128

NC = 2
NS = 16
NW = NC * NS
IBLK = ROWS // LANES
IB_PER_W = IBLK // NW
PAIRS = COLS // 2


def _sc_body(xf_hbm, w_hbm, out5_hbm, xblk,
             cidx0, cidx1, gbuf0, gbuf1, tbuf0, tbuf1,
             gsem0, gsem1, osem0, osem1):
    wid = lax.axis_index("s") * NC + lax.axis_index("c")
    iota = lax.iota(jnp.int32, 16)
    cidx = (cidx0, cidx1)
    gbuf = (gbuf0, gbuf1)
    tbuf = (tbuf0, tbuf1)
    gsem = (gsem0, gsem1)
    osem = (osem0, osem1)

    def build_cidx(s, j1):
        for k in range(8):
            flat = (k * 16 + iota) * LANES + j1
            v = plsc.load_gather(xblk, [flat])
            g = ((v & ~(TBLK - 1)) + ((v & (TBLK // 4 - 1)) << 2)
                 + ((v >> 11) & 3))
            cidx[s][pl.ds(k * 16, 16)] = g

    def fire_gather(s):
        pltpu.async_copy(w_hbm.at[cidx[s]], gbuf[s], gsem[s])

    def wait_gather(s):
        pltpu.make_async_copy(w_hbm.at[cidx[s]], gbuf[s], gsem[s]).wait()

    def transpose(s):
        def r_body(r, c3):
            rvec = jnp.full((16,), 0, jnp.int32) + r
            for h in range(2):
                vals = gbuf[s][r, pl.ds(16 * h, 16)]
                plsc.store_scatter(tbuf[s], [16 * h + iota, rvec], vals)
            return c3

        lax.fori_loop(0, LANES, r_body, 0)

    def fire_out(s, j1, c):
        for r in range(4):
            pltpu.async_copy(tbuf[s].at[pl.ds(r * 8, 8), pl.ds(0, LANES)],
                             out5_hbm.at[j1, r, c], osem[s])

    def drain_out(s):
        for r in range(4):
            pltpu.make_async_copy(
                tbuf[s].at[pl.ds(r * 8, 8), pl.ds(0, LANES)],
                out5_hbm.at[0, 0, 0], osem[s]).wait()

    def iblk_body(ib, carry):
        c = wid * IB_PER_W + ib
        pltpu.sync_copy(
            xf_hbm.at[pl.ds(c * LANES * LANES, LANES * LANES)], xblk)
        build_cidx(0, 0)
        fire_gather(0)

        def pair(t, carry2):
            j1a = 2 * t
            j1b = 2 * t + 1
            build_cidx(1, j1b)
            fire_gather(1)
            wait_gather(0)

            @pl.when(t > 0)
            def _():
                drain_out(0)

            transpose(0)
            fire_out(0, j1a, c)

            @pl.when(t < PAIRS - 1)
            def _():
                build_cidx(0, j1a + 2)
                fire_gather(0)

            wait_gather(1)

            @pl.when(t > 0)
            def _():
                drain_out(1)

            transpose(1)
            fire_out(1, j1b, c)
            return carry2

        lax.fori_loop(0, PAIRS, pair, 0)
        drain_out(0)
        drain_out(1)
        return carry

    lax.fori_loop(0, IB_PER_W, iblk_body, 0)


TBLK = 8192
TGRID = -(-NUM_ROWS_W // TBLK)
WROWS = TGRID * TBLK


def _tc_relayout_body(wt_ref, out_ref):
    q = TBLK // 4
    for m in range(4):
        out_ref[:, 32 * m:32 * (m + 1)] = (
            wt_ref[:, q * m:q * (m + 1)].T * SCALE)


def _relayout_weight(weight):
    w_rm = pl.pallas_call(
        _tc_relayout_body,
        grid=(TGRID,),
        in_specs=[pl.BlockSpec((DIM, TBLK), lambda i: (0, i))],
        out_specs=pl.BlockSpec((TBLK // 4, LANES), lambda i: (i, 0)),
        out_shape=jax.ShapeDtypeStruct((WROWS // 4, LANES), jnp.float32),
    )(weight.T)
    return w_rm.reshape(WROWS, DIM)


def kernel(x, weight):
    xf = jnp.pad(x.astype(jnp.int32), ((0, 0), (0, LANES - COLS))).reshape(-1)
    w_lin = _relayout_weight(weight)
    mesh = plsc.VectorSubcoreMesh(core_axis_name="c", subcore_axis_name="s")
    out5 = pl.kernel(
        _sc_body,
        out_type=jax.ShapeDtypeStruct((COLS, DIM // 8, IBLK, 8, LANES),
                                      jnp.float32),
        mesh=mesh,
        scratch_types=[
            pltpu.VMEM((LANES * LANES,), jnp.int32),
            pltpu.VMEM((LANES,), jnp.int32),
            pltpu.VMEM((LANES,), jnp.int32),
            pltpu.VMEM((LANES, DIM), jnp.float32),
            pltpu.VMEM((LANES, DIM), jnp.float32),
            pltpu.VMEM((DIM, LANES + 1), jnp.float32),
            pltpu.VMEM((DIM, LANES + 1), jnp.float32),
            pltpu.SemaphoreType.DMA,
            pltpu.SemaphoreType.DMA,
            pltpu.SemaphoreType.DMA,
            pltpu.SemaphoreType.DMA,
        ],
        compiler_params=pltpu.CompilerParams(use_tc_tiling_on_sc=False,
                                             needs_layout_passes=False),
    )(xf, w_lin)
    return out5.transpose((2, 4, 0, 1, 3)).reshape(ROWS, COLS, DIM)


# speedup vs baseline: 2.7960x; 1.0143x over previous
import jax
import jax.numpy as jnp
from jax import lax
from jax.experimental import pallas as pl
from jax.experimental.pallas import tpu as pltpu
from jax.experimental.pallas import tpu_sc as plsc

ROWS = 16384
NUM_ROWS_W = 1000000
COLS = 50
DIM = 32
SCALE = 10.0
LANES = 128

NC = 2
NS = 16
NW = NC * NS
IBLK = ROWS // LANES
IB_PER_W = IBLK // NW
PAIRS = COLS // 2


def _sc_body(xf_hbm, w_hbm, out5_hbm, xblk,
             cidx0, cidx1, gbuf0, gbuf1, tbuf0, tbuf1,
             gsem0, gsem1, osem0, osem1):
    wid = lax.axis_index("s") * NC + lax.axis_index("c")
    iota = lax.iota(jnp.int32, 16)
    cidx = (cidx0, cidx1)
    gbuf = (gbuf0, gbuf1)
    tbuf = (tbuf0, tbuf1)
    gsem = (gsem0, gsem1)
    osem = (osem0, osem1)

    def build_cidx(s, j1):
        for k in range(8):
            flat = (k * 16 + iota) * LANES + j1
            v = plsc.load_gather(xblk, [flat])
            g = ((v & ~(TBLK - 1)) + ((v & (TBLK // 4 - 1)) << 2)
                 + ((v >> 13) & 3))
            cidx[s][pl.ds(k * 16, 16)] = g

    def fire_gather(s):
        pltpu.async_copy(w_hbm.at[cidx[s]], gbuf[s], gsem[s])

    def wait_gather(s):
        pltpu.make_async_copy(w_hbm.at[cidx[s]], gbuf[s], gsem[s]).wait()

    def transpose(s):
        def r_body(r, c3):
            rvec = jnp.full((16,), 0, jnp.int32) + r
            for h in range(2):
                vals = gbuf[s][r, pl.ds(16 * h, 16)]
                plsc.store_scatter(tbuf[s], [16 * h + iota, rvec], vals)
            return c3

        lax.fori_loop(0, LANES, r_body, 0)

    def fire_out(s, j1, c):
        for r in range(4):
            pltpu.async_copy(tbuf[s].at[pl.ds(r * 8, 8), pl.ds(0, LANES)],
                             out5_hbm.at[j1, r, c], osem[s])

    def drain_out(s):
        for r in range(4):
            pltpu.make_async_copy(
                tbuf[s].at[pl.ds(r * 8, 8), pl.ds(0, LANES)],
                out5_hbm.at[0, 0, 0], osem[s]).wait()

    def iblk_body(ib, carry):
        c = wid * IB_PER_W + ib
        pltpu.sync_copy(
            xf_hbm.at[pl.ds(c * LANES * LANES, LANES * LANES)], xblk)
        build_cidx(0, 0)
        fire_gather(0)

        def pair(t, carry2):
            j1a = 2 * t
            j1b = 2 * t + 1
            build_cidx(1, j1b)
            fire_gather(1)
            wait_gather(0)

            @pl.when(t > 0)
            def _():
                drain_out(0)

            transpose(0)
            fire_out(0, j1a, c)

            @pl.when(t < PAIRS - 1)
            def _():
                build_cidx(0, j1a + 2)
                fire_gather(0)

            wait_gather(1)

            @pl.when(t > 0)
            def _():
                drain_out(1)

            transpose(1)
            fire_out(1, j1b, c)
            return carry2

        lax.fori_loop(0, PAIRS, pair, 0)
        drain_out(0)
        drain_out(1)
        return carry

    lax.fori_loop(0, IB_PER_W, iblk_body, 0)


TBLK = 32768
TGRID = -(-NUM_ROWS_W // TBLK)
WROWS = TGRID * TBLK


def _tc_relayout_body(wt_ref, out_ref):
    q = TBLK // 4
    for m in range(4):
        out_ref[:, 32 * m:32 * (m + 1)] = (
            wt_ref[:, q * m:q * (m + 1)].T * SCALE)


def _relayout_weight(weight):
    w_rm = pl.pallas_call(
        _tc_relayout_body,
        grid=(TGRID,),
        in_specs=[pl.BlockSpec((DIM, TBLK), lambda i: (0, i))],
        out_specs=pl.BlockSpec((TBLK // 4, LANES), lambda i: (i, 0)),
        out_shape=jax.ShapeDtypeStruct((WROWS // 4, LANES), jnp.float32),
    )(weight.T)
    return w_rm.reshape(WROWS, DIM)


def kernel(x, weight):
    xf = jnp.pad(x.astype(jnp.int32), ((0, 0), (0, LANES - COLS))).reshape(-1)
    w_lin = _relayout_weight(weight)
    mesh = plsc.VectorSubcoreMesh(core_axis_name="c", subcore_axis_name="s")
    out5 = pl.kernel(
        _sc_body,
        out_type=jax.ShapeDtypeStruct((COLS, DIM // 8, IBLK, 8, LANES),
                                      jnp.float32),
        mesh=mesh,
        scratch_types=[
            pltpu.VMEM((LANES * LANES,), jnp.int32),
            pltpu.VMEM((LANES,), jnp.int32),
            pltpu.VMEM((LANES,), jnp.int32),
            pltpu.VMEM((LANES, DIM), jnp.float32),
            pltpu.VMEM((LANES, DIM), jnp.float32),
            pltpu.VMEM((DIM, LANES + 1), jnp.float32),
            pltpu.VMEM((DIM, LANES + 1), jnp.float32),
            pltpu.SemaphoreType.DMA,
            pltpu.SemaphoreType.DMA,
            pltpu.SemaphoreType.DMA,
            pltpu.SemaphoreType.DMA,
        ],
        compiler_params=pltpu.CompilerParams(use_tc_tiling_on_sc=False,
                                             needs_layout_passes=False),
    )(xf, w_lin)
    return out5.transpose((2, 4, 0, 1, 3)).reshape(ROWS, COLS, DIM)
